# Initial kernel scaffold; baseline (speedup 1.0000x reference)
import jax
import jax.numpy as jnp
from jax.experimental import pallas as pl


def kernel(q, k, embeddings):
    raise NotImplementedError("write your pallas kernel here")


# speedup vs baseline: 6.5851x; 6.5851x over previous
import functools

import jax
import jax.numpy as jnp
from jax import lax
from jax.experimental import pallas as pl
from jax.experimental.pallas import tpu as pltpu
from jax.experimental.pallas import tpu_sc as plsc


@functools.lru_cache(maxsize=None)
def _build_sc_kernel(q_len, k_len, in_dim, out_dim):
    info = plsc.get_sparse_core_info()
    num_cores, num_subcores, lanes = (
        info.num_cores, info.num_subcores, info.num_lanes)
    num_workers = num_cores * num_subcores
    rows_per_worker = q_len // num_workers
    window = k_len + rows_per_worker - 1
    window_pad = ((window + lanes - 1) // lanes) * lanes
    max_index = (in_dim - 1) // 2

    mesh = plsc.VectorSubcoreMesh(core_axis_name="c", subcore_axis_name="s")

    @functools.partial(
        pl.kernel,
        mesh=mesh,
        compiler_params=pltpu.CompilerParams(use_tc_tiling_on_sc=False),
        out_type=jax.ShapeDtypeStruct((q_len, k_len, out_dim), jnp.float32),
        scratch_types=[
            pltpu.VMEM((window_pad,), jnp.int32),
            pltpu.VMEM((window_pad, out_dim), jnp.float32),
            pltpu.SemaphoreType.DMA,
        ],
    )
    def rel_pos_kernel(emb_hbm, out_hbm, idx_v, yw_v, sem):
        wid = lax.axis_index("s") * num_cores + lax.axis_index("c")
        i0 = wid * rows_per_worker

        def build_idx(t, carry):
            m = t * lanes + lax.iota(jnp.int32, lanes)
            v = (rows_per_worker - 1) + i0 - m
            v = jnp.clip(v, -max_index, max_index) + max_index
            idx_v[pl.ds(t * lanes, lanes)] = v
            return carry

        lax.fori_loop(0, window_pad // lanes, build_idx, 0)

        pltpu.async_copy(emb_hbm.at[idx_v], yw_v, sem).wait()

        def write_row(r, carry):
            pltpu.sync_copy(
                yw_v.at[pl.ds((rows_per_worker - 1) - r, k_len)],
                out_hbm.at[i0 + r],
            )
            return carry

        lax.fori_loop(0, rows_per_worker, write_row, 0)

    return rel_pos_kernel


def kernel(q, k, embeddings):
    q_len = q.shape[1]
    k_len = k.shape[1]
    in_dim, out_dim = embeddings.shape
    return _build_sc_kernel(q_len, k_len, in_dim, out_dim)(embeddings)
